# baseline (speedup 1.0000x reference)
import functools

import jax
import jax.numpy as jnp
from jax import lax
from jax.experimental import pallas as pl
from jax.experimental.pallas import tpu as pltpu
from jax.experimental.pallas import tpu_sc as plsc

_SC_ROWS = 512


@functools.cache
def _sc_head_copy(num_rows, dim, dtype):
    info = plsc.get_sparse_core_info()
    nw = info.num_cores * info.num_subcores
    assert num_rows % nw == 0, (num_rows, nw)
    rows_per_w = num_rows // nw
    mesh = plsc.VectorSubcoreMesh(core_axis_name="c", subcore_axis_name="s")

    @functools.partial(
        pl.kernel,
        mesh=mesh,
        out_type=jax.ShapeDtypeStruct((num_rows, dim), dtype),
        scratch_types=[pltpu.VMEM((rows_per_w, dim), dtype)],
    )
    def k(table_hbm, out_hbm, buf):
        wid = lax.axis_index("s") * info.num_cores + lax.axis_index("c")
        base = wid * rows_per_w
        pltpu.sync_copy(table_hbm.at[pl.ds(base, rows_per_w)], buf)
        pltpu.sync_copy(buf, out_hbm.at[pl.ds(base, rows_per_w)])

    return k


@functools.cache
def _tc_tail_copy(row0, num_rows, dim, dtype):
    blk = 256
    assert num_rows % blk == 0 and row0 % blk == 0

    def body(t_ref, o_ref):
        o_ref[...] = t_ref[...]

    return pl.pallas_call(
        body,
        grid=(num_rows // blk,),
        in_specs=[pl.BlockSpec((blk, dim), lambda i: (i + row0 // blk, 0))],
        out_specs=pl.BlockSpec((blk, dim), lambda i: (i, 0)),
        out_shape=jax.ShapeDtypeStruct((num_rows, dim), dtype),
    )


def kernel(seq_embeds, table):
    len_seq = seq_embeds.shape[-2]
    dim = table.shape[-1]
    head = _sc_head_copy(_SC_ROWS, dim, table.dtype)(table)
    tail = _tc_tail_copy(_SC_ROWS, len_seq - _SC_ROWS, dim, table.dtype)(table)
    pos_embeds = jnp.concatenate([head, tail], axis=0)
    if seq_embeds.ndim == 3:
        pos_embeds = pos_embeds[None]
    return pos_embeds


# speedup vs baseline: 1.2758x; 1.2758x over previous
import functools

import jax
import jax.numpy as jnp
from jax import lax
from jax.experimental import pallas as pl
from jax.experimental.pallas import tpu as pltpu
from jax.experimental.pallas import tpu_sc as plsc


@functools.cache
def _pos_embed_copy(num_rows, dim, dtype):
    info = plsc.get_sparse_core_info()
    nw = info.num_cores * info.num_subcores
    assert num_rows % nw == 0, (num_rows, nw)
    rows_per_w = num_rows // nw
    mesh = plsc.VectorSubcoreMesh(core_axis_name="c", subcore_axis_name="s")

    @functools.partial(
        pl.kernel,
        mesh=mesh,
        out_type=jax.ShapeDtypeStruct((num_rows, dim), dtype),
        scratch_types=[pltpu.VMEM((rows_per_w, dim), dtype)],
    )
    def k(table_hbm, out_hbm, buf):
        wid = lax.axis_index("s") * info.num_cores + lax.axis_index("c")
        base = wid * rows_per_w
        pltpu.sync_copy(table_hbm.at[pl.ds(base, rows_per_w)], buf)
        pltpu.sync_copy(buf, out_hbm.at[pl.ds(base, rows_per_w)])

    return k


def kernel(seq_embeds, table):
    len_seq = seq_embeds.shape[-2]
    pos_embeds = _pos_embed_copy(len_seq, table.shape[-1], table.dtype)(table)
    if seq_embeds.ndim == 3:
        pos_embeds = pos_embeds[None]
    return pos_embeds
